# baseline (speedup 1.0000x reference)
import functools

import jax
import jax.numpy as jnp
from jax import lax
from jax.experimental import pallas as pl
from jax.experimental.pallas import tpu as pltpu
from jax.experimental.pallas import tpu_sc as plsc

_V = 1000000
_D = 64
_B = 16384

_info = plsc.get_sparse_core_info()
_NC, _NS, _L = _info.num_cores, _info.num_subcores, _info.num_lanes
_NW = _NC * _NS
_SLAB = 512
_UNITS = 1954
_TAIL_U = 1953
_TAIL_OFF = _TAIL_U * _SLAB
_TAIL_W = _V - _TAIL_OFF
_UPW = 61
_LOC_CAP = 4096
_RING = 128


def kernel(labels, table, gamma, beta, null_emb):
    del null_emb

    table_t = table.T

    mesh = plsc.VectorSubcoreMesh(core_axis_name="c", subcore_axis_name="s")

    @functools.partial(
        pl.kernel,
        mesh=mesh,
        out_type=jax.ShapeDtypeStruct((_B * _D,), jnp.float32),
        scratch_types=[
            pltpu.VMEM((_B,), jnp.int32),
            pltpu.VMEM((_LOC_CAP,), jnp.int32),
            pltpu.VMEM((_LOC_CAP,), jnp.int32),
            pltpu.VMEM((_D, _SLAB), jnp.float32),
            pltpu.VMEM((_D, _SLAB), jnp.float32),
            pltpu.VMEM((_RING, _D), jnp.float32),
            pltpu.VMEM((_L,), jnp.int32),
            pltpu.VMEM((_L,), jnp.int32),
            pltpu.VMEM((_D,), jnp.float32),
            pltpu.VMEM((_D,), jnp.float32),
            pltpu.SemaphoreType.DMA,
            pltpu.SemaphoreType.DMA,
            pltpu.SemaphoreType.DMA,
        ],
        compiler_params=pltpu.CompilerParams(
            needs_layout_passes=False, use_tc_tiling_on_sc=True),
    )
    def _emb_ln(labels_h, tt_h, gamma_h, beta_h, out_h,
                lab_v, loc_r, loc_i, buf_a, buf_b, ring_v, tmp_r, tmp_i,
                gam_v, bet_v, sem_a, sem_b, sem_w):
        wid = lax.axis_index("s") * _NC + lax.axis_index("c")
        u0 = wid * _UPW + jnp.minimum(wid, 2)
        nu = _UPW + jnp.where(wid < 2, 1, 0)
        u_end = u0 + nu
        lo = u0 * _SLAB
        hi = u_end * _SLAB

        pltpu.sync_copy(gamma_h, gam_v)
        pltpu.sync_copy(beta_h, bet_v)
        pltpu.sync_copy(labels_h, lab_v)

        lane = lax.broadcasted_iota(jnp.int32, (_L,), 0)

        def fbody(k, cnt):
            lv = lab_v[pl.ds(k * _L, _L)]
            iv = lane + k * _L
            m = (lv >= lo) & (lv < hi)
            plsc.store_compressed(loc_r.at[pl.ds(cnt, _L)], lv, mask=m)
            plsc.store_compressed(loc_i.at[pl.ds(cnt, _L)], iv, mask=m)
            pc = plsc.all_reduce_population_count(m)
            return cnt + pc[0]

        n_loc = lax.fori_loop(0, _B // _L, fbody, jnp.int32(0))
        n_vregs = (n_loc + _L - 1) // _L

        inv_d = jnp.float32(1.0 / _D)
        nt = _D // _L
        gvs = [gam_v[pl.ds(t * _L, _L)] for t in range(nt)]
        bvs = [bet_v[pl.ds(t * _L, _L)] for t in range(nt)]
        cvecs = [lane + t * _L for t in range(nt)]

        def start_unit(u, buf, sem):
            act = u < u_end
            is_tail = u == _TAIL_U
            s0 = pl.multiple_of(u * _SLAB, _SLAB)

            @pl.when(act & jnp.logical_not(is_tail))
            def _():
                pltpu.make_async_copy(
                    tt_h.at[:, pl.ds(s0, _SLAB)], buf, sem).start()

            @pl.when(act & is_tail)
            def _():
                pltpu.make_async_copy(
                    tt_h.at[:, pl.ds(pl.multiple_of(u * _SLAB, 128), 128)],
                    buf.at[:, pl.ds(0, 128)], sem).start()

        def wait_unit(u, buf, sem):
            act = u < u_end
            is_tail = u == _TAIL_U

            @pl.when(act & jnp.logical_not(is_tail))
            def _():
                pltpu.make_async_copy(
                    tt_h.at[:, pl.ds(0, _SLAB)], buf, sem).wait()

            @pl.when(act & is_tail)
            def _():
                pltpu.make_async_copy(
                    tt_h.at[:, pl.ds(0, 128)],
                    buf.at[:, pl.ds(0, 128)], sem).wait()

        def process_unit(u, buf, cnt):
            act = u < u_end
            s0 = u * _SLAB
            s1 = s0 + _SLAB

            def svb(p, cnt):
                lv = loc_r[pl.ds(p * _L, _L)]
                iv = loc_i[pl.ds(p * _L, _L)]
                m = (lv >= s0) & (lv < s1)
                plsc.store_compressed(tmp_r.at[pl.ds(0, _L)], lv, mask=m)
                plsc.store_compressed(tmp_i.at[pl.ds(0, _L)], iv, mask=m)
                pc = plsc.all_reduce_population_count(m)

                def mbody(j, cnt):
                    jv = jnp.full((_L,), j, jnp.int32)
                    rv = plsc.load_gather(tmp_r, [jv])
                    pv = plsc.load_gather(tmp_i, [jv])
                    col = rv - s0
                    xs = [plsc.load_gather(buf, [cvecs[t], col])
                          for t in range(nt)]
                    s = (xs[0] + xs[1]) + (xs[2] + xs[3])
                    q = (xs[0] * xs[0] + xs[1] * xs[1]) + (
                        xs[2] * xs[2] + xs[3] * xs[3])
                    mean = jnp.sum(s, axis=0) * inv_d
                    var = jnp.sum(q, axis=0) * inv_d - mean * mean
                    vpe = var + jnp.float32(1e-5)
                    bits = lax.bitcast_convert_type(vpe, jnp.int32)
                    bits = jnp.int32(0x5F3759DF) - lax.shift_right_arithmetic(
                        bits, jnp.int32(1))
                    y = lax.bitcast_convert_type(bits, jnp.float32)
                    half = jnp.float32(0.5) * vpe
                    for _ in range(3):
                        y = y * (jnp.float32(1.5) - half * y * y)

                    slot = lax.rem(cnt, jnp.int32(_RING))

                    @pl.when(cnt >= _RING)
                    def _():
                        pltpu.make_async_copy(
                            ring_v.at[0], out_h.at[pl.ds(0, _D)],
                            sem_w).wait()

                    for t in range(nt):
                        ring_v[slot, pl.ds(t * _L, _L)] = (
                            (xs[t] - mean) * (y * gvs[t]) + bvs[t])
                    oi = pv[0]
                    pltpu.make_async_copy(
                        ring_v.at[slot], out_h.at[pl.ds(oi * _D, _D)],
                        sem_w).start()
                    return cnt + 1

                return lax.fori_loop(0, pc[0], mbody, cnt)

            bound = jnp.where(act, n_vregs, 0)
            return lax.fori_loop(0, bound, svb, cnt)

        start_unit(u0, buf_a, sem_a)

        def unit_pair(t, cnt):
            ua = u0 + 2 * t
            ub = ua + 1
            start_unit(ub, buf_b, sem_b)
            wait_unit(ua, buf_a, sem_a)
            cnt = process_unit(ua, buf_a, cnt)
            start_unit(ub + 1, buf_a, sem_a)
            wait_unit(ub, buf_b, sem_b)
            cnt = process_unit(ub, buf_b, cnt)
            return cnt

        cnt = lax.fori_loop(0, (_UPW + 2) // 2, unit_pair, jnp.int32(0))

        def dbody(_, c):
            pltpu.make_async_copy(
                ring_v.at[0], out_h.at[pl.ds(0, _D)], sem_w).wait()
            return c

        lax.fori_loop(0, jnp.minimum(cnt, _RING), dbody, jnp.int32(0))

    out1d = _emb_ln(labels, table_t, gamma, beta)

    @functools.partial(
        pl.kernel,
        mesh=mesh,
        out_type=jax.ShapeDtypeStruct((_D, _B), jnp.float32),
        scratch_types=[
            pltpu.VMEM((128 * _D,), jnp.float32),
            pltpu.VMEM((128 * _D,), jnp.float32),
            pltpu.VMEM((4, _D, 128), jnp.float32),
            pltpu.SemaphoreType.DMA,
            pltpu.SemaphoreType.DMA,
            pltpu.SemaphoreType.DMA,
        ],
        compiler_params=pltpu.CompilerParams(
            needs_layout_passes=False, use_tc_tiling_on_sc=True),
    )
    def _repack(flat_h, outt_h, in_a, in_b, tr_v, sem_a, sem_b, sem_o):
        wid = lax.axis_index("s") * _NC + lax.axis_index("c")
        npc = _B // 128 // _NW
        lane = lax.broadcasted_iota(jnp.int32, (_L,), 0)
        bufs = [(in_a, sem_a), (in_b, sem_b)]

        def start_in(c):
            i0 = (wid * npc + c) * 128
            buf, sem = bufs[c % 2]
            pltpu.make_async_copy(
                flat_h.at[pl.ds(i0 * _D, 128 * _D)], buf, sem).start()

        start_in(0)
        for c in range(npc):
            i0 = (wid * npc + c) * 128
            buf, sem = bufs[c % 2]
            if c + 1 < npc:
                start_in(c + 1)
            pltpu.make_async_copy(
                flat_h.at[pl.ds(0, 128 * _D)], buf, sem).wait()
            for d in range(_D):
                for g in range(128 // _L):
                    idx = (lane + g * _L) * _D + d
                    tr_v[c, d, pl.ds(g * _L, _L)] = plsc.load_gather(
                        buf, [idx])
            pltpu.make_async_copy(
                tr_v.at[c],
                outt_h.at[:, pl.ds(pl.multiple_of(i0, 128), 128)],
                sem_o).start()
        for c in range(npc):
            pltpu.make_async_copy(
                tr_v.at[0], outt_h.at[:, pl.ds(0, 128)], sem_o).wait()

    out_t = _repack(out1d)
    return out_t.T


# speedup vs baseline: 1.1480x; 1.1480x over previous
import functools

import jax
import jax.numpy as jnp
from jax import lax
from jax.experimental import pallas as pl
from jax.experimental.pallas import tpu as pltpu
from jax.experimental.pallas import tpu_sc as plsc

_V = 1000000
_D = 64
_B = 16384

_info = plsc.get_sparse_core_info()
_NC, _NS, _L = _info.num_cores, _info.num_subcores, _info.num_lanes
_NW = _NC * _NS
_SLAB = 512
_UNITS = 1954
_TAIL_U = 1953
_TAIL_OFF = _TAIL_U * _SLAB
_TAIL_W = _V - _TAIL_OFF
_UPW = 61
_LOC_CAP = 4096
_RING = 128


def kernel(labels, table, gamma, beta, null_emb):
    del null_emb

    table_t = table.T

    mesh = plsc.VectorSubcoreMesh(core_axis_name="c", subcore_axis_name="s")

    @functools.partial(
        pl.kernel,
        mesh=mesh,
        out_type=jax.ShapeDtypeStruct((_B * 128,), jnp.float32),
        scratch_types=[
            pltpu.VMEM((_B,), jnp.int32),
            pltpu.VMEM((_LOC_CAP,), jnp.int32),
            pltpu.VMEM((_LOC_CAP,), jnp.int32),
            pltpu.VMEM((_D, _SLAB), jnp.float32),
            pltpu.VMEM((_D, _SLAB), jnp.float32),
            pltpu.VMEM((_RING, _D), jnp.float32),
            pltpu.VMEM((_L,), jnp.int32),
            pltpu.VMEM((_L,), jnp.int32),
            pltpu.VMEM((_D,), jnp.float32),
            pltpu.VMEM((_D,), jnp.float32),
            pltpu.SemaphoreType.DMA,
            pltpu.SemaphoreType.DMA,
            pltpu.SemaphoreType.DMA,
        ],
        compiler_params=pltpu.CompilerParams(
            needs_layout_passes=False, use_tc_tiling_on_sc=True),
    )
    def _emb_ln(labels_h, tt_h, gamma_h, beta_h, out_h,
                lab_v, loc_r, loc_i, buf_a, buf_b, ring_v, tmp_r, tmp_i,
                gam_v, bet_v, sem_a, sem_b, sem_w):
        wid = lax.axis_index("s") * _NC + lax.axis_index("c")
        u0 = wid * _UPW + jnp.minimum(wid, 2)
        nu = _UPW + jnp.where(wid < 2, 1, 0)
        u_end = u0 + nu
        lo = u0 * _SLAB
        hi = u_end * _SLAB

        pltpu.sync_copy(gamma_h, gam_v)
        pltpu.sync_copy(beta_h, bet_v)
        pltpu.sync_copy(labels_h, lab_v)

        lane = lax.broadcasted_iota(jnp.int32, (_L,), 0)

        def fbody(k, cnt):
            lv = lab_v[pl.ds(k * _L, _L)]
            iv = lane + k * _L
            m = (lv >= lo) & (lv < hi)
            plsc.store_compressed(loc_r.at[pl.ds(cnt, _L)], lv, mask=m)
            plsc.store_compressed(loc_i.at[pl.ds(cnt, _L)], iv, mask=m)
            pc = plsc.all_reduce_population_count(m)
            return cnt + pc[0]

        n_loc = lax.fori_loop(0, _B // _L, fbody, jnp.int32(0))
        n_vregs = (n_loc + _L - 1) // _L

        inv_d = jnp.float32(1.0 / _D)
        nt = _D // _L
        gvs = [gam_v[pl.ds(t * _L, _L)] for t in range(nt)]
        bvs = [bet_v[pl.ds(t * _L, _L)] for t in range(nt)]
        cvecs = [lane + t * _L for t in range(nt)]

        def start_unit(u, buf, sem):
            act = u < u_end
            is_tail = u == _TAIL_U
            s0 = pl.multiple_of(u * _SLAB, _SLAB)

            @pl.when(act & jnp.logical_not(is_tail))
            def _():
                pltpu.make_async_copy(
                    tt_h.at[:, pl.ds(s0, _SLAB)], buf, sem).start()

            @pl.when(act & is_tail)
            def _():
                pltpu.make_async_copy(
                    tt_h.at[:, pl.ds(pl.multiple_of(u * _SLAB, 128), 128)],
                    buf.at[:, pl.ds(0, 128)], sem).start()

        def wait_unit(u, buf, sem):
            act = u < u_end
            is_tail = u == _TAIL_U

            @pl.when(act & jnp.logical_not(is_tail))
            def _():
                pltpu.make_async_copy(
                    tt_h.at[:, pl.ds(0, _SLAB)], buf, sem).wait()

            @pl.when(act & is_tail)
            def _():
                pltpu.make_async_copy(
                    tt_h.at[:, pl.ds(0, 128)],
                    buf.at[:, pl.ds(0, 128)], sem).wait()

        def process_unit(u, buf, cnt):
            act = u < u_end
            s0 = u * _SLAB
            s1 = s0 + _SLAB

            def svb(p, cnt):
                lv = loc_r[pl.ds(p * _L, _L)]
                iv = loc_i[pl.ds(p * _L, _L)]
                m = (lv >= s0) & (lv < s1)
                plsc.store_compressed(tmp_r.at[pl.ds(0, _L)], lv, mask=m)
                plsc.store_compressed(tmp_i.at[pl.ds(0, _L)], iv, mask=m)
                pc = plsc.all_reduce_population_count(m)

                def mbody(j, cnt):
                    jv = jnp.full((_L,), j, jnp.int32)
                    rv = plsc.load_gather(tmp_r, [jv])
                    pv = plsc.load_gather(tmp_i, [jv])
                    col = rv - s0
                    xs = [plsc.load_gather(buf, [cvecs[t], col])
                          for t in range(nt)]
                    s = (xs[0] + xs[1]) + (xs[2] + xs[3])
                    q = (xs[0] * xs[0] + xs[1] * xs[1]) + (
                        xs[2] * xs[2] + xs[3] * xs[3])
                    mean = jnp.sum(s, axis=0) * inv_d
                    var = jnp.sum(q, axis=0) * inv_d - mean * mean
                    vpe = var + jnp.float32(1e-5)
                    bits = lax.bitcast_convert_type(vpe, jnp.int32)
                    bits = jnp.int32(0x5F3759DF) - lax.shift_right_arithmetic(
                        bits, jnp.int32(1))
                    y = lax.bitcast_convert_type(bits, jnp.float32)
                    half = jnp.float32(0.5) * vpe
                    for _ in range(3):
                        y = y * (jnp.float32(1.5) - half * y * y)

                    slot = lax.rem(cnt, jnp.int32(_RING))

                    @pl.when(cnt >= _RING)
                    def _():
                        pltpu.make_async_copy(
                            ring_v.at[0], out_h.at[pl.ds(0, _D)],
                            sem_w).wait()

                    for t in range(nt):
                        ring_v[slot, pl.ds(t * _L, _L)] = (
                            (xs[t] - mean) * (y * gvs[t]) + bvs[t])
                    oi = pv[0]
                    pltpu.make_async_copy(
                        ring_v.at[slot], out_h.at[pl.ds(oi * 128, _D)],
                        sem_w).start()
                    return cnt + 1

                return lax.fori_loop(0, pc[0], mbody, cnt)

            bound = jnp.where(act, n_vregs, 0)
            return lax.fori_loop(0, bound, svb, cnt)

        start_unit(u0, buf_a, sem_a)

        def unit_pair(t, cnt):
            ua = u0 + 2 * t
            ub = ua + 1
            start_unit(ub, buf_b, sem_b)
            wait_unit(ua, buf_a, sem_a)
            cnt = process_unit(ua, buf_a, cnt)
            start_unit(ub + 1, buf_a, sem_a)
            wait_unit(ub, buf_b, sem_b)
            cnt = process_unit(ub, buf_b, cnt)
            return cnt

        cnt = lax.fori_loop(0, (_UPW + 2) // 2, unit_pair, jnp.int32(0))

        def dbody(_, c):
            pltpu.make_async_copy(
                ring_v.at[0], out_h.at[pl.ds(0, _D)], sem_w).wait()
            return c

        lax.fori_loop(0, jnp.minimum(cnt, _RING), dbody, jnp.int32(0))

    out1d = _emb_ln(labels, table_t, gamma, beta)

    x2 = out1d.reshape(_B, 128)

    def _tr_body(x_ref, o_ref):
        o_ref[...] = x_ref[:, :_D].T

    out_t = pl.pallas_call(
        _tr_body,
        grid=(_B // 512,),
        in_specs=[pl.BlockSpec((512, 128), lambda i: (i, 0))],
        out_specs=pl.BlockSpec((_D, 512), lambda i: (0, i)),
        out_shape=jax.ShapeDtypeStruct((_D, _B), jnp.float32),
    )(x2)
    return out_t.T


# speedup vs baseline: 1.2126x; 1.0562x over previous
import functools

import jax
import jax.numpy as jnp
from jax import lax
from jax.experimental import pallas as pl
from jax.experimental.pallas import tpu as pltpu
from jax.experimental.pallas import tpu_sc as plsc

_V = 1000000
_D = 64
_B = 16384

_info = plsc.get_sparse_core_info()
_NC, _NS, _L = _info.num_cores, _info.num_subcores, _info.num_lanes
_NW = _NC * _NS
_SLAB = 512
_UNITS = 1954
_TAIL_U = 1953
_TAIL_OFF = _TAIL_U * _SLAB
_TAIL_W = _V - _TAIL_OFF
_UPW = 61
_LOC_CAP = 4096
_RING = 128


def kernel(labels, table, gamma, beta, null_emb):
    del null_emb

    table_t = table.T

    mesh = plsc.VectorSubcoreMesh(core_axis_name="c", subcore_axis_name="s")

    @functools.partial(
        pl.kernel,
        mesh=mesh,
        out_type=jax.ShapeDtypeStruct((_B * _D,), jnp.float32),
        scratch_types=[
            pltpu.VMEM((_B,), jnp.int32),
            pltpu.VMEM((_LOC_CAP,), jnp.int32),
            pltpu.VMEM((_LOC_CAP,), jnp.int32),
            pltpu.VMEM((_D, _SLAB), jnp.float32),
            pltpu.VMEM((_D, _SLAB), jnp.float32),
            pltpu.VMEM((_RING, _D), jnp.float32),
            pltpu.VMEM((_L,), jnp.int32),
            pltpu.VMEM((_L,), jnp.int32),
            pltpu.VMEM((_D,), jnp.float32),
            pltpu.VMEM((_D,), jnp.float32),
            pltpu.SemaphoreType.DMA,
            pltpu.SemaphoreType.DMA,
            pltpu.SemaphoreType.DMA,
        ],
        compiler_params=pltpu.CompilerParams(
            needs_layout_passes=False, use_tc_tiling_on_sc=True),
    )
    def _emb_ln(labels_h, tt_h, gamma_h, beta_h, out_h,
                lab_v, loc_r, loc_i, buf_a, buf_b, ring_v, tmp_r, tmp_i,
                gam_v, bet_v, sem_a, sem_b, sem_w):
        wid = lax.axis_index("s") * _NC + lax.axis_index("c")
        u0 = wid * _UPW + jnp.minimum(wid, 2)
        nu = _UPW + jnp.where(wid < 2, 1, 0)
        u_end = u0 + nu
        lo = u0 * _SLAB
        hi = u_end * _SLAB

        pltpu.sync_copy(gamma_h, gam_v)
        pltpu.sync_copy(beta_h, bet_v)
        pltpu.sync_copy(labels_h, lab_v)

        lane = lax.broadcasted_iota(jnp.int32, (_L,), 0)

        def start_unit(u, buf, sem):
            act = u < u_end
            is_tail = u == _TAIL_U
            s0 = pl.multiple_of(u * _SLAB, _SLAB)

            @pl.when(act & jnp.logical_not(is_tail))
            def _():
                pltpu.make_async_copy(
                    tt_h.at[:, pl.ds(s0, _SLAB)], buf, sem).start()

            @pl.when(act & is_tail)
            def _():
                pltpu.make_async_copy(
                    tt_h.at[:, pl.ds(pl.multiple_of(u * _SLAB, 128), 128)],
                    buf.at[:, pl.ds(0, 128)], sem).start()

        def wait_unit(u, buf, sem):
            act = u < u_end
            is_tail = u == _TAIL_U

            @pl.when(act & jnp.logical_not(is_tail))
            def _():
                pltpu.make_async_copy(
                    tt_h.at[:, pl.ds(0, _SLAB)], buf, sem).wait()

            @pl.when(act & is_tail)
            def _():
                pltpu.make_async_copy(
                    tt_h.at[:, pl.ds(0, 128)],
                    buf.at[:, pl.ds(0, 128)], sem).wait()

        start_unit(u0, buf_a, sem_a)
        start_unit(u0 + 1, buf_b, sem_b)

        def fbody(k, cnt):
            lv = lab_v[pl.ds(k * _L, _L)]
            iv = lane + k * _L
            m = (lv >= lo) & (lv < hi)
            plsc.store_compressed(loc_r.at[pl.ds(cnt, _L)], lv, mask=m)
            plsc.store_compressed(loc_i.at[pl.ds(cnt, _L)], iv, mask=m)
            pc = plsc.all_reduce_population_count(m)
            return cnt + pc[0]

        n_loc = lax.fori_loop(0, _B // _L, fbody, jnp.int32(0))
        n_vregs = (n_loc + _L - 1) // _L

        inv_d = jnp.float32(1.0 / _D)
        nt = _D // _L
        gvs = [gam_v[pl.ds(t * _L, _L)] for t in range(nt)]
        bvs = [bet_v[pl.ds(t * _L, _L)] for t in range(nt)]
        cvecs = [lane + t * _L for t in range(nt)]

        def process_unit(u, buf, cnt):
            act = u < u_end
            s0 = u * _SLAB
            s1 = s0 + _SLAB

            def svb(p, cnt):
                lv = loc_r[pl.ds(p * _L, _L)]
                iv = loc_i[pl.ds(p * _L, _L)]
                m = (lv >= s0) & (lv < s1)
                plsc.store_compressed(tmp_r.at[pl.ds(0, _L)], lv, mask=m)
                plsc.store_compressed(tmp_i.at[pl.ds(0, _L)], iv, mask=m)
                pc = plsc.all_reduce_population_count(m)

                def mbody(j, cnt):
                    jv = jnp.full((_L,), j, jnp.int32)
                    rv = plsc.load_gather(tmp_r, [jv])
                    pv = plsc.load_gather(tmp_i, [jv])
                    col = rv - s0
                    xs = [plsc.load_gather(buf, [cvecs[t], col])
                          for t in range(nt)]
                    s = (xs[0] + xs[1]) + (xs[2] + xs[3])
                    q = (xs[0] * xs[0] + xs[1] * xs[1]) + (
                        xs[2] * xs[2] + xs[3] * xs[3])
                    mean = jnp.sum(s, axis=0) * inv_d
                    var = jnp.sum(q, axis=0) * inv_d - mean * mean
                    vpe = var + jnp.float32(1e-5)
                    bits = lax.bitcast_convert_type(vpe, jnp.int32)
                    bits = jnp.int32(0x5F3759DF) - lax.shift_right_arithmetic(
                        bits, jnp.int32(1))
                    y = lax.bitcast_convert_type(bits, jnp.float32)
                    half = jnp.float32(0.5) * vpe
                    for _ in range(3):
                        y = y * (jnp.float32(1.5) - half * y * y)

                    slot = lax.rem(cnt, jnp.int32(_RING))

                    @pl.when(cnt >= _RING)
                    def _():
                        pltpu.make_async_copy(
                            ring_v.at[0], out_h.at[pl.ds(0, _D)],
                            sem_w).wait()

                    for t in range(nt):
                        ring_v[slot, pl.ds(t * _L, _L)] = (
                            (xs[t] - mean) * (y * gvs[t]) + bvs[t])
                    oi = pv[0]
                    pltpu.make_async_copy(
                        ring_v.at[slot], out_h.at[pl.ds(oi * _D, _D)],
                        sem_w).start()
                    return cnt + 1

                return lax.fori_loop(0, pc[0], mbody, cnt)

            bound = jnp.where(act, n_vregs, 0)
            return lax.fori_loop(0, bound, svb, cnt)

        def unit_pair(t, cnt):
            ua = u0 + 2 * t
            ub = ua + 1
            wait_unit(ua, buf_a, sem_a)
            cnt = process_unit(ua, buf_a, cnt)
            start_unit(ua + 2, buf_a, sem_a)
            wait_unit(ub, buf_b, sem_b)
            cnt = process_unit(ub, buf_b, cnt)
            start_unit(ub + 2, buf_b, sem_b)
            return cnt

        cnt = lax.fori_loop(0, (_UPW + 2) // 2, unit_pair, jnp.int32(0))

        def dbody(_, c):
            pltpu.make_async_copy(
                ring_v.at[0], out_h.at[pl.ds(0, _D)], sem_w).wait()
            return c

        lax.fori_loop(0, jnp.minimum(cnt, _RING), dbody, jnp.int32(0))

    out1d = _emb_ln(labels, table_t, gamma, beta)
    return out1d.reshape(_B, _D)
